# baseline (speedup 1.0000x reference)
import jax
import jax.numpy as jnp
from jax import lax
from jax.experimental import pallas as pl
from jax.experimental.pallas import tpu as pltpu
from jax.experimental.pallas import tpu_sc as plsc

D = 64
V = 1000000
SCALE = 8.0
NC = 2
NS = 16
NW = NC * NS
B = 4096
L = 200
B_TOT = B * L
UROW = 128
UNITS = B_TOT // UROW
U_PER_W = UNITS // NW
CHUNK = 512
U_PER_CH = CHUNK // UROW
NCHUNK = U_PER_W // U_PER_CH
IDX_PER_W = U_PER_W * UROW
LANES = 16
TCB = 512


def _spread_body(in_ref, out_ref):
    blk = in_ref[...]
    out_ref[...] = jnp.reshape(
        jnp.transpose(jnp.reshape(blk, (D, TCB // 2, 2)), (1, 2, 0)),
        (TCB // 2, 2 * D)) * SCALE


def _spread(table_tr):
    grid = (V + TCB - 1) // TCB
    return pl.pallas_call(
        _spread_body,
        grid=(grid,),
        in_specs=[pl.BlockSpec((D, TCB), lambda i: (0, i))],
        out_specs=pl.BlockSpec((TCB // 2, 2 * D), lambda i: (i, 0)),
        out_shape=jax.ShapeDtypeStruct((V // 2, 2 * D), jnp.float32),
    )(table_tr)


def _emb_body(t2_hbm, idx_hbm, out_hbm,
              idx_all, rows0, rows1, tbuf, gsem0, gsem1, ssem):
    cid = lax.axis_index("c")
    sid = lax.axis_index("s")
    wid = sid * NC + cid
    rbufs = (rows0, rows1)
    gsems = (gsem0, gsem1)
    iota16 = lax.iota(jnp.int32, LANES)

    pltpu.sync_copy(idx_hbm.at[pl.ds(wid * IDX_PER_W, IDX_PER_W)], idx_all)

    def gather_descs(g, s):
        return [
            pltpu.make_async_copy(
                t2_hbm.at[idx_all.at[pl.ds(g * CHUNK + j * UROW, UROW)]],
                rbufs[s].at[pl.ds(j * UROW, UROW), :],
                gsems[s],
            )
            for j in range(U_PER_CH)
        ]

    def fire(g, s):
        for d_ in gather_descs(g, s):
            d_.start()

    def drain(g, s):
        for d_ in gather_descs(g, s):
            d_.wait()

    def unit_store_descs(uu):
        l = lax.div(uu, B // UROW)
        tj = lax.rem(uu, B // UROW)
        return [
            pltpu.make_async_copy(
                tbuf.at[pl.ds(ti * 8, 8), :],
                out_hbm.at[l, ti, tj, :, :],
                ssem,
            )
            for ti in range(8)
        ]

    def transform(u, s):
        rb = rbufs[s]

        @pl.loop(0, D)
        def _(d):
            col = jnp.full((LANES,), 0, jnp.int32) + d
            for bc in range(UROW // LANES):
                ridx = iota16 + (u * UROW + bc * LANES)
                vals = plsc.load_gather(rb, [ridx, col])
                tbuf[d, pl.ds(bc * LANES, LANES)] = vals

    def process_chunk(g, s):
        drain(g, s)
        for u in range(U_PER_CH):
            uu = wid * U_PER_W + g * U_PER_CH + u
            transform(u, s)
            descs = unit_store_descs(uu)
            for d_ in descs:
                d_.start()
            for d_ in descs:
                d_.wait()

    fire(0, 0)
    fire(1, 1)

    @pl.loop(0, NCHUNK - 2, step=2)
    def _(g0):
        for s in range(2):
            g = g0 + s
            process_chunk(g, s)
            fire(g + 2, s)

    for s in range(2):
        process_chunk(NCHUNK - 2 + s, s)


@jax.jit
def _emb_lookup(t2, idx1):
    mesh = plsc.VectorSubcoreMesh(core_axis_name="c", subcore_axis_name="s")
    f = pl.kernel(
        _emb_body,
        out_type=jax.ShapeDtypeStruct((L, 8, B // UROW, 8, UROW),
                                      jnp.float32),
        mesh=mesh,
        scratch_types=[
            pltpu.VMEM((IDX_PER_W,), jnp.int32),
            pltpu.VMEM((CHUNK, D), jnp.float32),
            pltpu.VMEM((CHUNK, D), jnp.float32),
            pltpu.VMEM((D, UROW), jnp.float32),
            pltpu.SemaphoreType.DMA,
            pltpu.SemaphoreType.DMA,
            pltpu.SemaphoreType.DMA,
        ],
        compiler_params=pltpu.CompilerParams(use_tc_tiling_on_sc=False,
                                             needs_layout_passes=False),
    )
    return f(t2, idx1)


def kernel(x, table):
    table_tr = jnp.transpose(table)
    t2 = _spread(table_tr).reshape(V, D)
    idx1 = jnp.transpose(x).reshape(-1)
    out5 = _emb_lookup(t2, idx1)
    return jnp.transpose(out5, (2, 4, 0, 1, 3)).reshape(B, L, D)


# speedup vs baseline: 3.7683x; 3.7683x over previous
import jax
import jax.numpy as jnp
from jax import lax
from jax.experimental import pallas as pl
from jax.experimental.pallas import tpu as pltpu
from jax.experimental.pallas import tpu_sc as plsc

D = 64
V = 1000000
SCALE = 8.0
NC = 2
NS = 16
NW = NC * NS
B = 4096
L = 200
B_TOT = B * L
UROW = 128
UNITS = B_TOT // UROW
U_PER_W = UNITS // NW
CHUNK = 512
U_PER_CH = CHUNK // UROW
NCHUNK = U_PER_W // U_PER_CH
IDX_PER_W = U_PER_W * UROW
LANES = 16
TCB = 512


def _spread_body(in_ref, eye_ref, out_ref):
    blk = in_ref[...]
    t = jax.lax.dot_general(
        blk, eye_ref[...],
        dimension_numbers=(((0,), (0,)), ((), ())),
        precision=jax.lax.Precision.HIGHEST,
        preferred_element_type=jnp.float32,
    )
    out_ref[...] = jnp.concatenate([t, t], axis=1)


def _spread(table_tr, eye8):
    grid = (V + TCB - 1) // TCB
    return pl.pallas_call(
        _spread_body,
        grid=(grid,),
        in_specs=[pl.BlockSpec((D, TCB), lambda i: (0, i)),
                  pl.BlockSpec((D, D), lambda i: (0, 0))],
        out_specs=pl.BlockSpec((TCB, 2 * D), lambda i: (i, 0)),
        out_shape=jax.ShapeDtypeStruct((V, 2 * D), jnp.float32),
    )(table_tr, eye8)


def _emb_body(t2_hbm, idx_hbm, out_hbm,
              idx_all, rows0, rows1, tb0, tb1, gsem0, gsem1, ssem):
    cid = lax.axis_index("c")
    sid = lax.axis_index("s")
    wid = sid * NC + cid
    rbufs = (rows0, rows1)
    tbufs = (tb0, tb1)
    gsems = (gsem0, gsem1)
    iota16 = lax.iota(jnp.int32, LANES)

    pltpu.sync_copy(idx_hbm.at[pl.ds(wid * IDX_PER_W, IDX_PER_W)], idx_all)

    @pl.loop(0, IDX_PER_W // LANES, unroll=8)
    def _(i):
        s = pl.multiple_of(i * LANES, 8)
        idx_all[pl.ds(s, LANES)] = idx_all[pl.ds(s, LANES)] * 2

    def gather_descs(g, s):
        return [
            pltpu.make_async_copy(
                t2_hbm.at[idx_all.at[pl.ds(g * CHUNK + j * UROW, UROW)]],
                rbufs[s].at[pl.ds(j * UROW, UROW), :],
                gsems[s],
            )
            for j in range(U_PER_CH)
        ]

    def fire(g, s):
        for d_ in gather_descs(g, s):
            d_.start()

    def drain(g, s):
        for d_ in gather_descs(g, s):
            d_.wait()

    def unit_store_descs(uu, tb):
        l = lax.div(uu, B // UROW)
        tj = lax.rem(uu, B // UROW)
        return [
            pltpu.make_async_copy(
                tb.at[pl.ds(ti * 8, 8), :],
                out_hbm.at[l, ti, tj, :, :],
                ssem,
            )
            for ti in range(8)
        ]

    def transform(u, s, tb):
        rb = rbufs[s]

        @pl.loop(0, UROW // LANES)
        def _(bc):
            ridx = iota16 + (u * UROW + bc * LANES)
            off = pl.multiple_of(bc * LANES, 16)

            @pl.loop(0, D, unroll=8)
            def _(d):
                col = jnp.full((LANES,), 0, jnp.int32) + d
                tb[d, pl.ds(off, LANES)] = (
                    plsc.load_gather(rb, [ridx, col]))

    def process_chunk(g, s):
        drain(g, s)
        pend = [None, None]
        for u in range(U_PER_CH):
            tb = tbufs[u % 2]
            if pend[u % 2] is not None:
                for d_ in pend[u % 2]:
                    d_.wait()
            transform(u, s, tb)
            uu = wid * U_PER_W + g * U_PER_CH + u
            descs = unit_store_descs(uu, tb)
            for d_ in descs:
                d_.start()
            pend[u % 2] = descs
        for p in pend:
            if p is not None:
                for d_ in p:
                    d_.wait()

    fire(0, 0)
    fire(1, 1)

    @pl.loop(0, NCHUNK - 2, step=2)
    def _(g0):
        for s in range(2):
            process_chunk(g0 + s, s)
            fire(g0 + s + 2, s)

    for s in range(2):
        process_chunk(NCHUNK - 2 + s, s)


@jax.jit
def _emb_lookup(t2, idx1):
    mesh = plsc.VectorSubcoreMesh(core_axis_name="c", subcore_axis_name="s")
    f = pl.kernel(
        _emb_body,
        out_type=jax.ShapeDtypeStruct((L, 8, B // UROW, 8, UROW),
                                      jnp.float32),
        mesh=mesh,
        scratch_types=[
            pltpu.VMEM((IDX_PER_W,), jnp.int32),
            pltpu.VMEM((CHUNK, D), jnp.float32),
            pltpu.VMEM((CHUNK, D), jnp.float32),
            pltpu.VMEM((D, UROW), jnp.float32),
            pltpu.VMEM((D, UROW), jnp.float32),
            pltpu.SemaphoreType.DMA,
            pltpu.SemaphoreType.DMA,
            pltpu.SemaphoreType.DMA,
        ],
        compiler_params=pltpu.CompilerParams(use_tc_tiling_on_sc=False,
                                             needs_layout_passes=False),
    )
    return f(t2, idx1)


def kernel(x, table):
    table_tr = jnp.transpose(table)
    eye8 = jnp.eye(D, dtype=jnp.float32) * SCALE
    t2 = _spread(table_tr, eye8).reshape(2 * V, D)
    idx1 = jnp.transpose(x).reshape(-1)
    out5 = _emb_lookup(t2, idx1)
    return jnp.transpose(out5, (2, 4, 0, 1, 3)).reshape(B, L, D)


# speedup vs baseline: 6.3764x; 1.6921x over previous
import jax
import jax.numpy as jnp
from jax import lax
from jax.experimental import pallas as pl
from jax.experimental.pallas import tpu as pltpu
from jax.experimental.pallas import tpu_sc as plsc

D = 64
SCALE = 8.0
NC = 2
NS = 16
NW = NC * NS
B_TOT = 4096 * 200
B_PER_W = B_TOT // NW
CHUNK = 256
NBUF = 2
NCHUNK = B_PER_W // CHUNK
GSPLIT = CHUNK // 128
LANES = 16
PROW = CHUNK // 2


def _emb_body(table_hbm, idx_hbm, out_hbm,
              idx0, idx1, rows0, rows1, ob0, ob1,
              gsem0, gsem1, ssem0, ssem1):
    idx_v = (idx0, idx1)
    rows_v = (rows0, rows1)
    obuf_v = (ob0, ob1)
    gsems = (gsem0, gsem1)
    ssems = (ssem0, ssem1)

    wid = lax.axis_index("s") * NC + lax.axis_index("c")
    base = wid * B_PER_W

    def gather_descs(b):
        return [
            pltpu.make_async_copy(
                table_hbm.at[idx_v[b].at[pl.ds(j * 128, 128)]],
                rows_v[b].at[pl.ds(j * 128, 128), :],
                gsems[b],
            )
            for j in range(GSPLIT)
        ]

    def fetch(g, b):
        pltpu.sync_copy(idx_hbm.at[pl.ds(base + g * CHUNK, CHUNK)],
                        idx_v[b])
        for d_ in gather_descs(b):
            d_.start()

    def drain_gather(b):
        for d_ in gather_descs(b):
            d_.wait()

    def store_desc(g, b):
        off = wid * (B_PER_W // 2) + g * PROW
        return pltpu.make_async_copy(
            obuf_v[b], out_hbm.at[pl.ds(off, PROW), :], ssems[b])

    def scale_pack(b):
        src = rows_v[b]
        dst = obuf_v[b]

        @pl.loop(0, PROW, unroll=4)
        def _(k):
            for par in range(2):
                for c in range(D // LANES):
                    dst[k, pl.ds(par * D + c * LANES, LANES)] = (
                        src[2 * k + par, pl.ds(c * LANES, LANES)] * SCALE)

    for b in range(NBUF):
        fetch(b, b)
    for b in range(NBUF):
        drain_gather(b)
        scale_pack(b)
        store_desc(b, b).start()
        fetch(b + NBUF, b)

    @pl.loop(NBUF, NCHUNK - NBUF, step=NBUF)
    def _(g0):
        for b in range(NBUF):
            g = g0 + b
            drain_gather(b)
            store_desc(g - NBUF, b).wait()
            scale_pack(b)
            store_desc(g, b).start()
            fetch(g + NBUF, b)

    for b in range(NBUF):
        g = NCHUNK - NBUF + b
        drain_gather(b)
        store_desc(g - NBUF, b).wait()
        scale_pack(b)
        store_desc(g, b).start()
    for b in range(NBUF):
        store_desc(NCHUNK - NBUF + b, b).wait()


@jax.jit
def _emb_lookup(table, idx):
    mesh = plsc.VectorSubcoreMesh(core_axis_name="c", subcore_axis_name="s")
    f = pl.kernel(
        _emb_body,
        out_type=jax.ShapeDtypeStruct((B_TOT // 2, 2 * D), jnp.float32),
        mesh=mesh,
        scratch_types=[
            pltpu.VMEM((CHUNK,), jnp.int32),
            pltpu.VMEM((CHUNK,), jnp.int32),
            pltpu.VMEM((CHUNK, D), jnp.float32),
            pltpu.VMEM((CHUNK, D), jnp.float32),
            pltpu.VMEM((PROW, 2 * D), jnp.float32),
            pltpu.VMEM((PROW, 2 * D), jnp.float32),
            pltpu.SemaphoreType.DMA,
            pltpu.SemaphoreType.DMA,
            pltpu.SemaphoreType.DMA,
            pltpu.SemaphoreType.DMA,
        ],
        compiler_params=pltpu.CompilerParams(use_tc_tiling_on_sc=False),
    )
    return f(table, idx)


def kernel(x, table):
    idx = x.reshape(-1)
    out2 = _emb_lookup(table, idx)
    return out2.reshape(x.shape + (D,))


# speedup vs baseline: 8.2585x; 1.2952x over previous
import jax
import jax.numpy as jnp
from jax import lax
from jax.experimental import pallas as pl
from jax.experimental.pallas import tpu as pltpu
from jax.experimental.pallas import tpu_sc as plsc

D = 64
SCALE = 8.0
NC = 2
NS = 16
NW = NC * NS
B_TOT = 4096 * 200
B_PER_W = B_TOT // NW
CHUNK = 512
NBUF = 2
NCHUNK = B_PER_W // CHUNK
GSPLIT = CHUNK // 128
LANES = 16


def _emb_body(table_hbm, idx_hbm, out_hbm,
              idx0, idx1, rows0, rows1, gsem0, gsem1, ssem0, ssem1):
    idx_v = (idx0, idx1)
    rows_v = (rows0, rows1)
    gsems = (gsem0, gsem1)
    ssems = (ssem0, ssem1)

    wid = lax.axis_index("s") * NC + lax.axis_index("c")
    base = wid * B_PER_W

    def gather_descs(b):
        return [
            pltpu.make_async_copy(
                table_hbm.at[idx_v[b].at[pl.ds(j * 128, 128)]],
                rows_v[b].at[pl.ds(j * 128, 128), :],
                gsems[b],
            )
            for j in range(GSPLIT)
        ]

    def fetch(g, b):
        off = base + g * CHUNK
        pltpu.sync_copy(idx_hbm.at[pl.ds(off, CHUNK)], idx_v[b])
        for d_ in gather_descs(b):
            d_.start()

    def drain_gather(b):
        for d_ in gather_descs(b):
            d_.wait()

    def store_desc(g, b):
        off = base + g * CHUNK
        return pltpu.make_async_copy(
            rows_v[b], out_hbm.at[pl.ds(off, CHUNK), :], ssems[b])

    def scale(b):
        r = rows_v[b]

        @pl.loop(0, CHUNK, unroll=8)
        def _(i):
            for j in range(D // LANES):
                sl = (i, pl.ds(j * LANES, LANES))
                r[sl] = r[sl] * SCALE

    for b in range(NBUF):
        fetch(b, b)

    @pl.loop(0, NCHUNK - NBUF, step=NBUF)
    def _(g0):
        for b in range(NBUF):
            drain_gather(b)
            scale(b)
            store_desc(g0 + b, b).start()
        for b in range(NBUF):
            store_desc(g0 + b, b).wait()
            fetch(g0 + b + NBUF, b)

    for b in range(NBUF):
        g = NCHUNK - NBUF + b
        drain_gather(b)
        scale(b)
        store_desc(g, b).start()
    for b in range(NBUF):
        store_desc(NCHUNK - NBUF + b, b).wait()


@jax.jit
def _emb_lookup(table, idx):
    mesh = plsc.VectorSubcoreMesh(core_axis_name="c", subcore_axis_name="s")
    f = pl.kernel(
        _emb_body,
        out_type=jax.ShapeDtypeStruct((B_TOT, D), jnp.float32),
        mesh=mesh,
        scratch_types=[
            pltpu.VMEM((CHUNK,), jnp.int32),
            pltpu.VMEM((CHUNK,), jnp.int32),
            pltpu.VMEM((CHUNK, D), jnp.float32),
            pltpu.VMEM((CHUNK, D), jnp.float32),
            pltpu.SemaphoreType.DMA,
            pltpu.SemaphoreType.DMA,
            pltpu.SemaphoreType.DMA,
            pltpu.SemaphoreType.DMA,
        ],
        compiler_params=pltpu.CompilerParams(use_tc_tiling_on_sc=False),
    )
    return f(table, idx)


def kernel(x, table):
    idx = x.reshape(-1)
    out = _emb_lookup(table, idx)
    return out.reshape(x.shape + (D,))
